# baseline (speedup 1.0000x reference)
import jax
import jax.numpy as jnp
from jax import lax
from jax.experimental import pallas as pl
from jax.experimental.pallas import tpu as pltpu
from jax.experimental.pallas import tpu_sc as plsc

EMBED = 16
REGS = 1e-05
B = 16384
NROWS = 1000000
NC, NS, L = 2, 16, 16
NW = NC * NS
BPW = B // NW
NBLK = BPW // L


def _softplus_neg_abs(l):
    t = jnp.exp(-jnp.abs(l))
    s = t / (t + 2.0)
    s2 = s * s
    return 2.0 * s * (1.0 + s2 * (1.0 / 3.0 + s2 * (0.2 + s2 * (1.0 / 7.0))))


def _disc_kernel(user, pos, neg, uemb_t, iemb_t, bias, out,
                 idx_u, idx_p, idx_n, u_cols, p_cols, n_cols,
                 b_p, b_n, stage, sem, bsem):
    wid = lax.axis_index("s") * NC + lax.axis_index("c")
    base = wid * BPW

    h0 = pltpu.async_copy(user.at[pl.ds(base, BPW)], idx_u, sem)
    h1 = pltpu.async_copy(pos.at[pl.ds(base, BPW)], idx_p, sem)
    h2 = pltpu.async_copy(neg.at[pl.ds(base, BPW)], idx_n, sem)
    h0.wait()
    h1.wait()
    h2.wait()

    handles = []
    for c in range(BPW // 128):
        sl = pl.ds(c * 128, 128)
        for d in range(EMBED):
            handles.append(pltpu.async_copy(
                uemb_t.at[d].at[idx_u.at[sl]], u_cols.at[d].at[sl], sem))
            handles.append(pltpu.async_copy(
                iemb_t.at[d].at[idx_p.at[sl]], p_cols.at[d].at[sl], sem))
            handles.append(pltpu.async_copy(
                iemb_t.at[d].at[idx_n.at[sl]], n_cols.at[d].at[sl], sem))
        handles.append(pltpu.async_copy(bias.at[idx_p.at[sl]], b_p.at[sl], bsem))
        handles.append(pltpu.async_copy(bias.at[idx_n.at[sl]], b_n.at[sl], bsem))
    for h in handles:
        h.wait()

    lanes = lax.iota(jnp.int32, L)
    zero = jnp.zeros((L,), jnp.float32)

    def block(j, carry):
        cls_acc, su, sp_, sn = carry
        sl = pl.ds(j * L, L)
        dp = zero
        dn = zero
        for d in range(EMBED):
            u = u_cols[d, sl]
            p = p_cols[d, sl]
            n = n_cols[d, sl]
            dp = dp + u * p
            dn = dn + u * n
            su = su + u * u
            sp_ = sp_ + p * p
            sn = sn + n * n
        lp = dp + b_p[sl]
        ln = dn + b_n[sl]
        pos_t = jnp.maximum(lp, 0.0) - lp + _softplus_neg_abs(lp)
        neg_t = jnp.maximum(ln, 0.0) + _softplus_neg_abs(ln)
        return (cls_acc + pos_t + neg_t, su, sp_, sn)

    cls_acc, su, sp_, sn = lax.fori_loop(
        0, NBLK, block, (zero, zero, zero, zero), unroll=2)

    cls_s = jnp.sum(cls_acc)
    sq_s = jnp.sum(2.0 * su + sp_ + sn)
    stage[...] = jnp.where(lanes == 0, cls_s,
                           jnp.where(lanes == 1, sq_s, 0.0))
    pltpu.sync_copy(stage, out.at[pl.ds(wid * L, L)])


@jax.jit
def kernel(user, pos, neg, user_embedding, item_embedding, bias):
    mesh = plsc.VectorSubcoreMesh(
        core_axis_name="c", subcore_axis_name="s",
        num_cores=NC, num_subcores=NS)
    k = pl.kernel(
        _disc_kernel,
        out_type=jax.ShapeDtypeStruct((NW * L,), jnp.float32),
        mesh=mesh,
        compiler_params=pltpu.CompilerParams(
            needs_layout_passes=False, use_tc_tiling_on_sc=False),
        scratch_types=[
            pltpu.VMEM((BPW,), jnp.int32),
            pltpu.VMEM((BPW,), jnp.int32),
            pltpu.VMEM((BPW,), jnp.int32),
            pltpu.VMEM((EMBED, BPW), jnp.float32),
            pltpu.VMEM((EMBED, BPW), jnp.float32),
            pltpu.VMEM((EMBED, BPW), jnp.float32),
            pltpu.VMEM((BPW,), jnp.float32),
            pltpu.VMEM((BPW,), jnp.float32),
            pltpu.VMEM((L,), jnp.float32),
            pltpu.SemaphoreType.DMA,
            pltpu.SemaphoreType.DMA,
        ],
    )
    part = k(user.astype(jnp.int32), pos.astype(jnp.int32),
             neg.astype(jnp.int32),
             user_embedding.T, item_embedding.T, bias)
    part = part.reshape(NW, L)
    cls_loss = jnp.sum(part[:, 0]) / B
    reg_loss = jnp.float32(REGS * 0.5 / B) * jnp.sum(part[:, 1])
    return (cls_loss, reg_loss)
